# baseline (speedup 1.0000x reference)
import functools

import jax
import jax.numpy as jnp
from jax import lax
from jax.experimental import pallas as pl
from jax.experimental.pallas import tpu as pltpu
from jax.experimental.pallas import tpu_sc as plsc

B = 64
N = 576
D = 768
FLAT = N * D
OUT_FLAT = (N + 1) * D

NC = 2
NS = 16
NW = NC * NS
BPW = B // NW

CH_ROWS = 64
F = CH_ROWS * D
NCH = FLAT // F
LANES = 16
UNROLL = 8


_mesh = plsc.VectorSubcoreMesh(core_axis_name="c", subcore_axis_name="s")


@functools.partial(
    pl.kernel,
    mesh=_mesh,
    out_type=jax.ShapeDtypeStruct((B, OUT_FLAT), jnp.float32),
    scratch_types=[
        pltpu.VMEM((F,), jnp.float32),
        pltpu.VMEM((F,), jnp.float32),
        pltpu.VMEM((D,), jnp.float32),
    ],
)
def _encode(patch_hbm, pos_hbm, out_hbm, pbuf, qbuf, row0):
    wid = lax.axis_index("s") * NC + lax.axis_index("c")
    b0 = wid * BPW

    pltpu.sync_copy(pos_hbm.at[pl.ds(0, D)], row0)
    for bb in range(BPW):
        pltpu.sync_copy(row0, out_hbm.at[b0 + bb, pl.ds(0, D)])

    def chunk_body(c, _):
        off = c * F
        pltpu.sync_copy(pos_hbm.at[pl.ds(D + off, F)], qbuf)

        def batch_body(bb, _):
            b = b0 + bb
            pltpu.sync_copy(patch_hbm.at[b, pl.ds(off, F)], pbuf)

            def add_body(i, _):
                base = i * (LANES * UNROLL)
                for u in range(UNROLL):
                    s = base + u * LANES
                    pbuf[pl.ds(s, LANES)] = (
                        pbuf[pl.ds(s, LANES)] + qbuf[pl.ds(s, LANES)]
                    )
                return 0

            lax.fori_loop(0, F // (LANES * UNROLL), add_body, 0)
            pltpu.sync_copy(pbuf, out_hbm.at[b, pl.ds(D + off, F)])
            return 0

        lax.fori_loop(0, BPW, batch_body, 0)
        return 0

    lax.fori_loop(0, NCH, chunk_body, 0)


def kernel(patch, pos_emb):
    patch_f = patch.reshape(B, FLAT)
    pos_f = pos_emb.reshape(OUT_FLAT)
    out = _encode(patch_f, pos_f)
    return out.reshape(B, N + 1, D)
